# baseline (speedup 1.0000x reference)
import functools

import jax
import jax.numpy as jnp
from jax import lax
from jax.experimental import pallas as pl
from jax.experimental.pallas import tpu as pltpu
from jax.experimental.pallas import tpu_sc as plsc

N = 10000
E = 160000
DIM = 256
H = 128
NP = 10240
SLICE = 640
EPT = 10032
EPAD = EPT * 16
EBUF = EPAD + 128
CHUNK = 48
P1C = 528
FCH = 8

_SC_PARAMS = pltpu.CompilerParams(needs_layout_passes=False)
_mesh = plsc.VectorSubcoreMesh(core_axis_name="c", subcore_axis_name="s")


def _mm_body(x_ref, w_ref, ws_ref, h_ref, hs_ref):
    xb = x_ref[...]
    dn = (((1,), (1,)), ((), ()))
    h_ref[0] = lax.dot_general(xb, w_ref[0], dn, preferred_element_type=jnp.float32)
    hs_ref[0] = lax.dot_general(xb, ws_ref[0], dn, preferred_element_type=jnp.float32)


def _tc_project(x, w2, ws2):
    nb = 10
    rb = NP // nb
    return pl.pallas_call(
        _mm_body,
        grid=(2, nb),
        in_specs=[
            pl.BlockSpec((rb, DIM), lambda j, i: (i, 0)),
            pl.BlockSpec((1, H, DIM), lambda j, i: (j, 0, 0)),
            pl.BlockSpec((1, H, DIM), lambda j, i: (j, 0, 0)),
        ],
        out_specs=[
            pl.BlockSpec((1, rb, H), lambda j, i: (j, i, 0)),
            pl.BlockSpec((1, rb, H), lambda j, i: (j, i, 0)),
        ],
        out_shape=[
            jax.ShapeDtypeStruct((2, NP, H), jnp.float32),
            jax.ShapeDtypeStruct((2, NP, H), jnp.float32),
        ],
    )(x, w2, ws2)


def _sigmoid16(v):
    return 1.0 / (1.0 + jnp.exp(-v))


@functools.partial(
    pl.kernel,
    mesh=_mesh,
    compiler_params=_SC_PARAMS,
    out_type=jax.ShapeDtypeStruct((2, NP, H), jnp.float32),
    scratch_types=[
        pltpu.VMEM((16,), jnp.int32),
        pltpu.VMEM((16,), jnp.int32),
        pltpu.VMEM((SLICE,), jnp.float32),
        pltpu.VMEM((SLICE,), jnp.float32),
        pltpu.VMEM((SLICE,), jnp.float32),
        pltpu.VMEM((16,), jnp.float32),
        pltpu.VMEM((16,), jnp.float32),
        pltpu.VMEM_SHARED((16, NP), jnp.float32),
        pltpu.VMEM_SHARED((16, NP), jnp.float32),
        pltpu.VMEM_SHARED((EBUF,), jnp.float32),
    ],
)
def _sc_message(row_hbm, gidx0_hbm, sw_hbm, rep_a_hbm, rep_b_hbm,
                ns_hbm, reps_hbm, st_hbm, en_hbm, h2_hbm, hs2_hbm, out2,
                sbuf, ebuf, deg_sl, inv_sl, reps_sl, iva, ivb,
                stg_sn, stg_dg, coeff_sp):
    c = lax.axis_index("c")
    s = lax.axis_index("s")
    cNP = c * NP
    i16 = lax.iota(jnp.int32, 16)

    def pass1(ra, rb, nsb, snl, dgl, rsl, csl, swl, cof):
        pltpu.sync_copy(rep_a_hbm, ra)
        pltpu.sync_copy(rep_b_hbm, rb)
        pltpu.sync_copy(ns_hbm, nsb)

        zf = jnp.zeros((16,), jnp.float32)

        def zero_np(u, _):
            snl[pl.ds(u * 16, 16)] = zf
            dgl[pl.ds(u * 16, 16)] = zf
            return 0

        lax.fori_loop(0, NP // 16, zero_np, 0)

        for ech in range(EPT // P1C):
            e0 = s * EPT + ech * P1C
            pltpu.sync_copy(row_hbm.at[pl.ds(e0, P1C)], rsl)
            pltpu.sync_copy(gidx0_hbm.at[pl.ds(e0, P1C)], csl)
            pltpu.sync_copy(sw_hbm.at[pl.ds(e0, P1C)], swl)

            def edge_group(u, _):
                off = u * 16
                r16 = rsl[pl.ds(off, 16)]
                c16 = csl[pl.ds(off, 16)]
                sw16 = swl[pl.ds(off, 16)]
                ga = plsc.load_gather(ra, [r16])
                gb = plsc.load_gather(rb, [c16])
                nn = plsc.load_gather(nsb, [c16])
                gate = _sigmoid16(ga + gb)
                tn = 1.0 - 2.0 / (jnp.exp(2.0 * nn) + 1.0)
                cof[pl.ds(off, 16)] = sw16 * gate * tn
                plsc.addupdate_scatter(snl, [r16], sw16)
                plsc.addupdate_scatter(dgl, [r16], gate)
                return 0

            lax.fori_loop(0, P1C // 16, edge_group, 0)
            pltpu.sync_copy(cof, coeff_sp.at[pl.ds(e0, P1C)])

        pltpu.sync_copy(snl, stg_sn.at[s])
        pltpu.sync_copy(dgl, stg_dg.at[s])
        plsc.subcore_barrier()

        def combine(tmp):
            for ho, hs_ in ((0, 384), (384, 256)):
                nb = s * SLICE + ho
                for p in range(16):
                    pltpu.sync_copy(stg_sn.at[p, pl.ds(nb, hs_)],
                                    tmp.at[p, pl.ds(0, hs_)])

                def red_sn(g, _):
                    o = g * 16
                    acc = tmp[0, pl.ds(o, 16)]
                    for p in range(1, 16):
                        acc = acc + tmp[p, pl.ds(o, 16)]
                    inv_sl[pl.ds(ho + o, 16)] = 1.0 / (acc + 1e-6)
                    return 0

                lax.fori_loop(0, hs_ // 16, red_sn, 0)

                for p in range(16):
                    pltpu.sync_copy(stg_dg.at[p, pl.ds(nb, hs_)],
                                    tmp.at[p, pl.ds(0, hs_)])

                def red_dg(g, _):
                    o = g * 16
                    acc = tmp[0, pl.ds(o, 16)]
                    for p in range(1, 16):
                        acc = acc + tmp[p, pl.ds(o, 16)]
                    deg_sl[pl.ds(ho + o, 16)] = acc
                    return 0

                lax.fori_loop(0, hs_ // 16, red_dg, 0)

        pl.run_scoped(
            combine,
            pltpu.VMEM((16, 384), jnp.float32),
        )

    pl.run_scoped(
        pass1,
        pltpu.VMEM((NP,), jnp.float32),
        pltpu.VMEM((NP,), jnp.float32),
        pltpu.VMEM((NP,), jnp.float32),
        pltpu.VMEM((NP,), jnp.float32),
        pltpu.VMEM((NP,), jnp.float32),
        pltpu.VMEM((P1C,), jnp.int32),
        pltpu.VMEM((P1C,), jnp.int32),
        pltpu.VMEM((P1C,), jnp.float32),
        pltpu.VMEM((P1C,), jnp.float32),
    )

    def pass2(acc, gbuf0, gbuf1, rbuf0, rbuf1, gxbuf0, gxbuf1, cbuf0, cbuf1,
              si0, si1, sg0, sg1, so):
        pltpu.sync_copy(st_hbm, sbuf)
        pltpu.sync_copy(en_hbm, ebuf)
        pltpu.sync_copy(reps_hbm.at[pl.ds(s * SLICE, SLICE)], reps_sl)

        start = jnp.sum(jnp.where(i16 == s, sbuf[...], 0))
        end = jnp.sum(jnp.where(i16 == s, ebuf[...], 0))
        abase = (start // 8) * 8
        nch = (end - abase + (CHUNK - 1)) // CHUNK
        nmax = jnp.maximum(nch - 1, 0)

        zf = jnp.zeros((16,), jnp.float32)

        def zero_acc(u, _):
            acc[pl.ds(u * 16, 16)] = zf
            return 0

        lax.fori_loop(0, SLICE * H // 16, zero_acc, 0)

        nb = s * SLICE

        def chunk(i, _):
            base = abase + i * CHUNK
            pltpu.sync_copy(row_hbm.at[pl.ds(base, CHUNK)], rbuf0)
            pltpu.sync_copy(gidx0_hbm.at[pl.ds(base, CHUNK)], gxbuf0)
            pltpu.sync_copy(coeff_sp.at[pl.ds(base, CHUNK)], cbuf0)
            for g in range(CHUNK // 16):
                go = g * 16
                gxbuf0[pl.ds(go, 16)] = gxbuf0[pl.ds(go, 16)] + cNP
            pltpu.sync_copy(h2_hbm.at[gxbuf0], gbuf0)

            for g in range(CHUNK // 16):
                eo = g * 16
                r16 = rbuf0[pl.ds(eo, 16)]
                lr16 = jnp.minimum(jnp.maximum(r16 - nb, 0), SLICE - 1)
                co16 = cbuf0[pl.ds(eo, 16)] * plsc.load_gather(inv_sl, [lr16])
                eidx = base + eo + i16
                ok = (eidx >= start) & (eidx < end)
                co16 = jnp.where(ok, co16, 0.0)
                lr = lr16 * H
                e16 = i16 + eo

                def colm(m, _):
                    v = plsc.load_gather(gbuf0, [e16, jnp.full((16,), m, jnp.int32)])
                    plsc.addupdate_scatter(acc, [lr + m], v * co16)
                    return 0

                lax.fori_loop(0, H, colm, 0)
            return 0

        lax.fori_loop(0, nch, chunk, 0)

        NCHF = SLICE // 16

        def fbody(ch, _):
            hs_wait_gb = gbuf0
            pltpu.sync_copy(hs2_hbm.at[pl.ds(cNP + nb + ch * 16, 16)],
                            hs_wait_gb.at[pl.ds(16, 16)])
            lo = ch * 16
            d16 = deg_sl[pl.ds(lo, 16)]
            iva[...] = 1.0 / (d16 + 1e-6)
            ivb[...] = _sigmoid16(reps_sl[pl.ds(lo, 16)])

            def frow(k, _):
                kk = jnp.full((16,), k, jnp.int32)
                dsp = plsc.load_gather(iva, [kk])
                gsp = plsc.load_gather(ivb, [kk])
                ab = (lo + k) * H
                for m in range(H // 16):
                    ci = i16 + m * 16
                    a = plsc.load_gather(acc, [ab + ci])
                    hv = plsc.load_gather(hs_wait_gb, [kk + 16, ci])
                    o = a * dsp + gsp * hv
                    o = jnp.where(o >= 0.0, o, o * 0.01)
                    plsc.store_scatter(hs_wait_gb, [kk, ci], o)
                return 0

            lax.fori_loop(0, 16, frow, 0)
            pltpu.sync_copy(hs_wait_gb.at[pl.ds(0, 16)],
                            out2.at[c, pl.ds(nb + lo, 16)])
            return 0

        lax.fori_loop(0, NCHF, fbody, 0)

    pl.run_scoped(
        pass2,
        pltpu.VMEM((SLICE * H,), jnp.float32),
        pltpu.VMEM((CHUNK, H), jnp.float32),
        pltpu.VMEM((CHUNK, H), jnp.float32),
        pltpu.VMEM((CHUNK,), jnp.int32),
        pltpu.VMEM((CHUNK,), jnp.int32),
        pltpu.VMEM((CHUNK,), jnp.int32),
        pltpu.VMEM((CHUNK,), jnp.int32),
        pltpu.VMEM((CHUNK,), jnp.float32),
        pltpu.VMEM((CHUNK,), jnp.float32),
        pltpu.SemaphoreType.DMA,
        pltpu.SemaphoreType.DMA,
        pltpu.SemaphoreType.DMA,
        pltpu.SemaphoreType.DMA,
        pltpu.SemaphoreType.DMA,
    )


def kernel(x, edge_index, sim_weight, rep, node_signal, W, W_self,
           alpha, beta, alpha_self, temp):
    row = edge_index[0].astype(jnp.int32)
    col = edge_index[1].astype(jnp.int32)
    sw = sim_weight.astype(jnp.float32)

    order = jnp.argsort(row)
    npad = EBUF - E
    row_p = jnp.concatenate([row[order], jnp.full((npad,), NP - 1, jnp.int32)])
    col_p = jnp.concatenate([col[order], jnp.zeros((npad,), jnp.int32)])
    sw_p = jnp.concatenate([sw[order], jnp.zeros((npad,), jnp.float32)])
    starts = jnp.searchsorted(
        row_p[:EPAD], jnp.arange(16, dtype=jnp.int32) * SLICE).astype(jnp.int32)
    ends = jnp.concatenate([starts[1:], jnp.array([EPAD], jnp.int32)])

    pad_n = NP - N
    zn = jnp.zeros((pad_n,), jnp.float32)
    rep_a = jnp.concatenate([(alpha / temp) * rep, zn])
    rep_b = jnp.concatenate([(beta / temp) * rep, zn])
    ns_p = jnp.concatenate([node_signal, zn])
    reps = jnp.concatenate([(alpha_self / temp) * rep, zn])

    w2 = W.reshape(2, H, DIM)
    ws2 = W_self.reshape(2, H, DIM)
    x_p = jnp.concatenate([x, jnp.zeros((pad_n, DIM), jnp.float32)])
    h2, hs2 = _tc_project(x_p, w2, ws2)
    h2f = h2.reshape(2 * NP, H)
    hs2f = hs2.reshape(2 * NP, H)

    out2 = _sc_message(row_p, col_p, sw_p, rep_a, rep_b, ns_p,
                       reps, starts, ends, h2f, hs2f)
    return jnp.concatenate([out2[0, :N], out2[1, :N]], axis=1)


# speedup vs baseline: 1.0304x; 1.0304x over previous
import functools

import jax
import jax.numpy as jnp
from jax import lax
from jax.experimental import pallas as pl
from jax.experimental.pallas import tpu as pltpu
from jax.experimental.pallas import tpu_sc as plsc

N = 10000
E = 160000
DIM = 256
H = 128
NP = 10240
SLICE = 640
EPT = 10032
EPAD = EPT * 16
EBUF = EPAD + 128
CHUNK = 80
P1C = 528
FCH = 8

_SC_PARAMS = pltpu.CompilerParams(needs_layout_passes=False)
_mesh = plsc.VectorSubcoreMesh(core_axis_name="c", subcore_axis_name="s")


def _mm_body(x_ref, w_ref, ws_ref, h_ref, hs_ref):
    xb = x_ref[...]
    dn = (((1,), (1,)), ((), ()))
    h_ref[0] = lax.dot_general(xb, w_ref[0], dn, preferred_element_type=jnp.float32)
    hs_ref[0] = lax.dot_general(xb, ws_ref[0], dn, preferred_element_type=jnp.float32)


def _tc_project(x, w2, ws2):
    nb = 10
    rb = NP // nb
    return pl.pallas_call(
        _mm_body,
        grid=(2, nb),
        in_specs=[
            pl.BlockSpec((rb, DIM), lambda j, i: (i, 0)),
            pl.BlockSpec((1, H, DIM), lambda j, i: (j, 0, 0)),
            pl.BlockSpec((1, H, DIM), lambda j, i: (j, 0, 0)),
        ],
        out_specs=[
            pl.BlockSpec((1, rb, H), lambda j, i: (j, i, 0)),
            pl.BlockSpec((1, rb, H), lambda j, i: (j, i, 0)),
        ],
        out_shape=[
            jax.ShapeDtypeStruct((2, NP, H), jnp.float32),
            jax.ShapeDtypeStruct((2, NP, H), jnp.float32),
        ],
    )(x, w2, ws2)


def _sigmoid16(v):
    return 1.0 / (1.0 + jnp.exp(-v))


@functools.partial(
    pl.kernel,
    mesh=_mesh,
    compiler_params=_SC_PARAMS,
    out_type=jax.ShapeDtypeStruct((2, NP, H), jnp.float32),
    scratch_types=[
        pltpu.VMEM((16,), jnp.int32),
        pltpu.VMEM((16,), jnp.int32),
        pltpu.VMEM((SLICE,), jnp.float32),
        pltpu.VMEM((SLICE,), jnp.float32),
        pltpu.VMEM((SLICE,), jnp.float32),
        pltpu.VMEM((16,), jnp.float32),
        pltpu.VMEM((16,), jnp.float32),
        pltpu.VMEM_SHARED((16, NP), jnp.float32),
        pltpu.VMEM_SHARED((16, NP), jnp.float32),
        pltpu.VMEM_SHARED((EBUF,), jnp.float32),
    ],
)
def _sc_message(row_hbm, gidx0_hbm, rc_hbm, sw_hbm, rep_a_hbm, rep_b_hbm,
                ns_hbm, reps_hbm, st_hbm, en_hbm, h2_hbm, hs2_hbm, out2,
                sbuf, ebuf, deg_sl, inv_sl, reps_sl, iva, ivb,
                stg_sn, stg_dg, coeff_sp):
    c = lax.axis_index("c")
    s = lax.axis_index("s")
    cNP = c * NP
    i16 = lax.iota(jnp.int32, 16)

    def pass1(ra, rb, nsb, snl, dgl, rsl, csl, swl, cof):
        pltpu.sync_copy(rep_a_hbm, ra)
        pltpu.sync_copy(rep_b_hbm, rb)
        pltpu.sync_copy(ns_hbm, nsb)

        zf = jnp.zeros((16,), jnp.float32)

        def zero_np(u, _):
            snl[pl.ds(u * 16, 16)] = zf
            dgl[pl.ds(u * 16, 16)] = zf
            return 0

        lax.fori_loop(0, NP // 16, zero_np, 0)

        for ech in range(EPT // P1C):
            e0 = s * EPT + ech * P1C
            pltpu.sync_copy(row_hbm.at[pl.ds(e0, P1C)], rsl)
            pltpu.sync_copy(gidx0_hbm.at[pl.ds(e0, P1C)], csl)
            pltpu.sync_copy(sw_hbm.at[pl.ds(e0, P1C)], swl)

            def edge_group(u, _):
                off = u * 16
                r16 = rsl[pl.ds(off, 16)]
                c16 = csl[pl.ds(off, 16)]
                sw16 = swl[pl.ds(off, 16)]
                ga = plsc.load_gather(ra, [r16])
                gb = plsc.load_gather(rb, [c16])
                nn = plsc.load_gather(nsb, [c16])
                gate = _sigmoid16(ga + gb)
                tn = 1.0 - 2.0 / (jnp.exp(2.0 * nn) + 1.0)
                cof[pl.ds(off, 16)] = sw16 * gate * tn
                plsc.addupdate_scatter(snl, [r16], sw16)
                plsc.addupdate_scatter(dgl, [r16], gate)
                return 0

            lax.fori_loop(0, P1C // 16, edge_group, 0)
            pltpu.sync_copy(cof, coeff_sp.at[pl.ds(e0, P1C)])

        pltpu.sync_copy(snl, stg_sn.at[s])
        pltpu.sync_copy(dgl, stg_dg.at[s])
        plsc.subcore_barrier()

        def combine(tmp):
            for ho, hs_ in ((0, 384), (384, 256)):
                nb = s * SLICE + ho
                for p in range(16):
                    pltpu.sync_copy(stg_sn.at[p, pl.ds(nb, hs_)],
                                    tmp.at[p, pl.ds(0, hs_)])

                def red_sn(g, _):
                    o = g * 16
                    acc = tmp[0, pl.ds(o, 16)]
                    for p in range(1, 16):
                        acc = acc + tmp[p, pl.ds(o, 16)]
                    inv_sl[pl.ds(ho + o, 16)] = 1.0 / (acc + 1e-6)
                    return 0

                lax.fori_loop(0, hs_ // 16, red_sn, 0)

                for p in range(16):
                    pltpu.sync_copy(stg_dg.at[p, pl.ds(nb, hs_)],
                                    tmp.at[p, pl.ds(0, hs_)])

                def red_dg(g, _):
                    o = g * 16
                    acc = tmp[0, pl.ds(o, 16)]
                    for p in range(1, 16):
                        acc = acc + tmp[p, pl.ds(o, 16)]
                    deg_sl[pl.ds(ho + o, 16)] = acc
                    return 0

                lax.fori_loop(0, hs_ // 16, red_dg, 0)

        pl.run_scoped(
            combine,
            pltpu.VMEM((16, 384), jnp.float32),
        )

    pl.run_scoped(
        pass1,
        pltpu.VMEM((NP,), jnp.float32),
        pltpu.VMEM((NP,), jnp.float32),
        pltpu.VMEM((NP,), jnp.float32),
        pltpu.VMEM((NP,), jnp.float32),
        pltpu.VMEM((NP,), jnp.float32),
        pltpu.VMEM((P1C,), jnp.int32),
        pltpu.VMEM((P1C,), jnp.int32),
        pltpu.VMEM((P1C,), jnp.float32),
        pltpu.VMEM((P1C,), jnp.float32),
    )

    def pass2(acc, gbuf0, rcbuf, rbuf0, gxbuf0, cbuf0):
        pltpu.sync_copy(st_hbm, sbuf)
        pltpu.sync_copy(en_hbm, ebuf)
        pltpu.sync_copy(reps_hbm.at[pl.ds(s * SLICE, SLICE)], reps_sl)

        start = jnp.sum(jnp.where(i16 == s, sbuf[...], 0))
        end = jnp.sum(jnp.where(i16 == s, ebuf[...], 0))
        abase = (start // 8) * 8
        nch = (end - abase + (CHUNK - 1)) // CHUNK
        nmax = jnp.maximum(nch - 1, 0)

        zf = jnp.zeros((16,), jnp.float32)

        def zero_acc(u, _):
            acc[pl.ds(u * 16, 16)] = zf
            return 0

        lax.fori_loop(0, SLICE * H // 16, zero_acc, 0)

        nb = s * SLICE

        def chunk(i, _):
            base = abase + i * CHUNK
            pltpu.sync_copy(rc_hbm.at[pl.ds(base * 2, CHUNK * 2)], rcbuf)
            pltpu.sync_copy(coeff_sp.at[pl.ds(base, CHUNK)], cbuf0)
            for g in range(CHUNK // 16):
                go = g * 16
                ei = (go + i16) * 2
                rbuf0[pl.ds(go, 16)] = plsc.load_gather(rcbuf, [ei])
                gxbuf0[pl.ds(go, 16)] = plsc.load_gather(rcbuf, [ei + 1]) + cNP
            pltpu.sync_copy(h2_hbm.at[gxbuf0], gbuf0)

            for g in range(CHUNK // 16):
                eo = g * 16
                r16 = rbuf0[pl.ds(eo, 16)]
                lr16 = jnp.minimum(jnp.maximum(r16 - nb, 0), SLICE - 1)
                co16 = cbuf0[pl.ds(eo, 16)] * plsc.load_gather(inv_sl, [lr16])
                eidx = base + eo + i16
                ok = (eidx >= start) & (eidx < end)
                co16 = jnp.where(ok, co16, 0.0)
                lr = lr16 * H
                e16 = i16 + eo

                def colm(m, _):
                    v = plsc.load_gather(gbuf0, [e16, jnp.full((16,), m, jnp.int32)])
                    plsc.addupdate_scatter(acc, [lr + m], v * co16)
                    return 0

                lax.fori_loop(0, H, colm, 0)
            return 0

        lax.fori_loop(0, nch, chunk, 0)

        NCHF = SLICE // 16

        def fbody(ch, _):
            hs_wait_gb = gbuf0
            pltpu.sync_copy(hs2_hbm.at[pl.ds(cNP + nb + ch * 16, 16)],
                            hs_wait_gb.at[pl.ds(16, 16)])
            lo = ch * 16
            d16 = deg_sl[pl.ds(lo, 16)]
            iva[...] = 1.0 / (d16 + 1e-6)
            ivb[...] = _sigmoid16(reps_sl[pl.ds(lo, 16)])

            def frow(k, _):
                kk = jnp.full((16,), k, jnp.int32)
                dsp = plsc.load_gather(iva, [kk])
                gsp = plsc.load_gather(ivb, [kk])
                ab = (lo + k) * H
                for m in range(H // 16):
                    ci = i16 + m * 16
                    a = plsc.load_gather(acc, [ab + ci])
                    hv = plsc.load_gather(hs_wait_gb, [kk + 16, ci])
                    o = a * dsp + gsp * hv
                    o = jnp.where(o >= 0.0, o, o * 0.01)
                    plsc.store_scatter(hs_wait_gb, [kk, ci], o)
                return 0

            lax.fori_loop(0, 16, frow, 0)
            pltpu.sync_copy(hs_wait_gb.at[pl.ds(0, 16)],
                            out2.at[c, pl.ds(nb + lo, 16)])
            return 0

        lax.fori_loop(0, NCHF, fbody, 0)

    pl.run_scoped(
        pass2,
        pltpu.VMEM((SLICE * H,), jnp.float32),
        pltpu.VMEM((CHUNK, H), jnp.float32),
        pltpu.VMEM((CHUNK * 2,), jnp.int32),
        pltpu.VMEM((CHUNK,), jnp.int32),
        pltpu.VMEM((CHUNK,), jnp.int32),
        pltpu.VMEM((CHUNK,), jnp.float32),
    )


def kernel(x, edge_index, sim_weight, rep, node_signal, W, W_self,
           alpha, beta, alpha_self, temp):
    row = edge_index[0].astype(jnp.int32)
    col = edge_index[1].astype(jnp.int32)
    sw = sim_weight.astype(jnp.float32)

    order = jnp.argsort(row)
    npad = EBUF - E
    row_p = jnp.concatenate([row[order], jnp.full((npad,), NP - 1, jnp.int32)])
    col_p = jnp.concatenate([col[order], jnp.zeros((npad,), jnp.int32)])
    sw_p = jnp.concatenate([sw[order], jnp.zeros((npad,), jnp.float32)])
    starts = jnp.searchsorted(
        row_p[:EPAD], jnp.arange(16, dtype=jnp.int32) * SLICE).astype(jnp.int32)
    ends = jnp.concatenate([starts[1:], jnp.array([EPAD], jnp.int32)])

    pad_n = NP - N
    zn = jnp.zeros((pad_n,), jnp.float32)
    rep_a = jnp.concatenate([(alpha / temp) * rep, zn])
    rep_b = jnp.concatenate([(beta / temp) * rep, zn])
    ns_p = jnp.concatenate([node_signal, zn])
    reps = jnp.concatenate([(alpha_self / temp) * rep, zn])

    w2 = W.reshape(2, H, DIM)
    ws2 = W_self.reshape(2, H, DIM)
    x_p = jnp.concatenate([x, jnp.zeros((pad_n, DIM), jnp.float32)])
    h2, hs2 = _tc_project(x_p, w2, ws2)
    h2f = h2.reshape(2 * NP, H)
    hs2f = hs2.reshape(2 * NP, H)

    rc_p = jnp.stack([row_p, col_p], axis=1).reshape(-1)
    out2 = _sc_message(row_p, col_p, rc_p, sw_p, rep_a, rep_b, ns_p,
                       reps, starts, ends, h2f, hs2f)
    return jnp.concatenate([out2[0, :N], out2[1, :N]], axis=1)
